# baseline (speedup 1.0000x reference)
import numpy as np

import jax
import jax.numpy as jnp
from jax import lax
from jax.experimental import pallas as pl
from jax.experimental.pallas import tpu as pltpu
from jax.experimental.pallas import tpu_sc as plsc

NC, NS = 2, 16
NW = NC * NS
OUT_D = 10
PAD_D = 17
L = 16

VOCAB = 1000
BATCH, SEQ = 4096, 200
TOTAL = BATCH * SEQ
B_PER_W = BATCH // NW
CB = 4
N_CHUNKS = B_PER_W // CB
CROWS = CB * SEQ
GROUPS = CROWS // L


def _fuse_table_body(emb_ref, w_ref, b_ref, out_ref):
    out_ref[...] = (
        jnp.dot(emb_ref[...], w_ref[...], preferred_element_type=jnp.float32)
        + b_ref[...]
    )


def _pad_wb(W, b):
    wp = jnp.zeros((W.shape[0], PAD_D), jnp.float32).at[:, :OUT_D].set(W)
    bp = jnp.zeros((1, PAD_D), jnp.float32).at[0, :OUT_D].set(b)
    return wp, bp


def _gather_body(fused_hbm, idx_hbm, out_hbm, table_v, idx_v, out_v, sem):
    wid = lax.axis_index("s") * NC + lax.axis_index("c")
    bbase = wid * B_PER_W

    pltpu.sync_copy(fused_hbm, table_v)

    iota = lax.iota(jnp.int32, L)
    bvec, svec = [], []
    for t in range(GROUPS):
        q = iota + t * L
        bq = q // SEQ
        bvec.append(bq)
        svec.append(q - bq * SEQ)
    col_sel = [jnp.full((L,), c, jnp.int32) for c in range(OUT_D)]

    def chunk(g, carry):
        boff = bbase + g * CB
        pltpu.sync_copy(idx_hbm.at[pl.ds(boff * SEQ, CROWS)], idx_v)

        for t in range(GROUPS):
            rows = idx_v[pl.ds(t * L, L)] * PAD_D
            for c in range(OUT_D):
                vals = plsc.load_gather(table_v, [rows + c])
                plsc.store_scatter(out_v, [bvec[t], svec[t], col_sel[c]], vals)

        pltpu.sync_copy(out_v, out_hbm.at[pl.ds(boff, CB)])
        return carry

    lax.fori_loop(0, N_CHUNKS, chunk, 0)


def kernel(batch, emb_table, W, b):
    wp, bp = _pad_wb(W, b)
    fused = pl.pallas_call(
        _fuse_table_body,
        out_shape=jax.ShapeDtypeStruct((VOCAB, PAD_D), jnp.float32),
    )(emb_table, wp, bp).reshape(VOCAB * PAD_D)

    idx = batch.reshape(TOTAL)

    mesh = plsc.VectorSubcoreMesh(core_axis_name="c", subcore_axis_name="s")
    out = pl.kernel(
        _gather_body,
        out_type=jax.ShapeDtypeStruct((BATCH, SEQ, OUT_D), jnp.float32),
        mesh=mesh,
        scratch_types=[
            pltpu.VMEM((VOCAB * PAD_D,), jnp.float32),
            pltpu.VMEM((CROWS,), jnp.int32),
            pltpu.VMEM((CB, SEQ, OUT_D), jnp.float32),
            pltpu.SemaphoreType.DMA,
        ],
        compiler_params=pltpu.CompilerParams(
            use_tc_tiling_on_sc=False, needs_layout_passes=False
        ),
    )(fused, idx)

    return out


# speedup vs baseline: 1.3065x; 1.3065x over previous
import jax
import jax.numpy as jnp
from jax import lax
from jax.experimental import pallas as pl
from jax.experimental.pallas import tpu as pltpu
from jax.experimental.pallas import tpu_sc as plsc

NC, NS = 2, 16
NW = NC * NS
OUT_D = 10
PAD_D = 17
L = 16

VOCAB = 1000
BATCH, SEQ = 4096, 200
TOTAL = BATCH * SEQ
N_PER_W = TOTAL // NW
CHUNK = 400
N_CHUNKS = N_PER_W // CHUNK
GROUPS = CHUNK // L


def _fuse_table_body(emb_ref, w_ref, b_ref, out_ref):
    out_ref[...] = (
        jnp.dot(emb_ref[...], w_ref[...], preferred_element_type=jnp.float32)
        + b_ref[...]
    )


def _pad_wb(W, b):
    wp = jnp.zeros((W.shape[0], PAD_D), jnp.float32).at[:, :OUT_D].set(W)
    bp = jnp.zeros((1, PAD_D), jnp.float32).at[0, :OUT_D].set(b)
    return wp, bp


def _gather_body(fused_hbm, idx_hbm, out_hbm, table_v, idx_v, out_a, out_b, sem):
    wid = lax.axis_index("s") * NC + lax.axis_index("c")
    base = wid * N_PER_W
    bufs = (out_a, out_b)

    pltpu.sync_copy(fused_hbm, table_v)

    iota = lax.iota(jnp.int32, L)
    col_sel = [jnp.full((L,), c, jnp.int32) for c in range(OUT_D)]
    orow = [t * L + iota for t in range(GROUPS)]

    def do_chunk(g, buf):
        off = base + g * CHUNK
        pltpu.sync_copy(idx_hbm.at[pl.ds(off, CHUNK)], idx_v)
        for t in range(GROUPS):
            rows = idx_v[pl.ds(t * L, L)] * PAD_D
            for c in range(OUT_D):
                vals = plsc.load_gather(table_v, [rows + c])
                plsc.store_scatter(buf, [orow[t], col_sel[c]], vals)
        pltpu.async_copy(buf, out_hbm.at[pl.ds(off, CHUNK)], sem)

    def drain(buf):
        pltpu.make_async_copy(buf, out_hbm.at[pl.ds(base, CHUNK)], sem).wait()

    do_chunk(0, out_a)
    do_chunk(1, out_b)

    def steady(g2, carry):
        for par in range(2):
            drain(bufs[par])
            do_chunk(g2 * 2 + par, bufs[par])
        return carry

    lax.fori_loop(1, N_CHUNKS // 2, steady, 0)
    drain(out_a)
    drain(out_b)


def kernel(batch, emb_table, W, b):
    wp, bp = _pad_wb(W, b)
    fused = pl.pallas_call(
        _fuse_table_body,
        out_shape=jax.ShapeDtypeStruct((VOCAB, PAD_D), jnp.float32),
    )(emb_table, wp, bp).reshape(VOCAB * PAD_D)

    idx = batch.reshape(TOTAL)

    mesh = plsc.VectorSubcoreMesh(core_axis_name="c", subcore_axis_name="s")
    flat = pl.kernel(
        _gather_body,
        out_type=jax.ShapeDtypeStruct((TOTAL, OUT_D), jnp.float32),
        mesh=mesh,
        scratch_types=[
            pltpu.VMEM((VOCAB * PAD_D,), jnp.float32),
            pltpu.VMEM((CHUNK,), jnp.int32),
            pltpu.VMEM((CHUNK, OUT_D), jnp.float32),
            pltpu.VMEM((CHUNK, OUT_D), jnp.float32),
            pltpu.SemaphoreType.DMA,
        ],
        compiler_params=pltpu.CompilerParams(
            use_tc_tiling_on_sc=True, needs_layout_passes=False
        ),
    )(fused, idx)

    return flat.reshape(BATCH, SEQ, OUT_D)
